# baseline (speedup 1.0000x reference)
import functools

import jax
import jax.numpy as jnp
from jax import lax
from jax.experimental import pallas as pl
from jax.experimental.pallas import tpu as pltpu
from jax.experimental.pallas import tpu_sc as plsc

_N = 10000
_E = 320000
_IN_DIM = 128
_OUT_DIM = 128
_DIM = 256
_HID = 256
_LAYERS = 4

_BN = 1000


def _kernel_a(x_ref, w1_ref, b1_ref, wg_ref, asrc_ref, adst_ref,
              x0_ref, h_ref, aa_ref):
    x0 = jnp.dot(x_ref[...], w1_ref[...],
                 preferred_element_type=jnp.float32) + b1_ref[...]
    x0_ref[...] = x0
    cols = []
    for l in range(_LAYERS):
        h = jnp.dot(x0, wg_ref[l], preferred_element_type=jnp.float32)
        h_ref[l, 0] = h[:, :128]
        h_ref[l, 1] = h[:, 128:]
        cols.append(jnp.sum(h * asrc_ref[l][None, :], axis=1))
        cols.append(jnp.sum(h * adst_ref[l][None, :], axis=1))
    aa_ref[...] = jnp.stack(cols, axis=1)


def _run_a(x, lin1_W, lin1_b, gat_W, att_src, att_dst):
    nblk = _N // _BN
    full = lambda *shape: pl.BlockSpec(shape, lambda i: (0,) * len(shape))
    return pl.pallas_call(
        _kernel_a,
        grid=(nblk,),
        in_specs=[
            pl.BlockSpec((_BN, _IN_DIM), lambda i: (i, 0)),
            full(_IN_DIM, _DIM),
            full(1, _DIM),
            full(_LAYERS, _DIM, _DIM),
            full(_LAYERS, _DIM),
            full(_LAYERS, _DIM),
        ],
        out_specs=[
            pl.BlockSpec((_BN, _DIM), lambda i: (i, 0)),
            pl.BlockSpec((_LAYERS, 2, _BN, 128), lambda i: (0, 0, i, 0)),
            pl.BlockSpec((_BN, 2 * _LAYERS), lambda i: (i, 0)),
        ],
        out_shape=[
            jax.ShapeDtypeStruct((_N, _DIM), jnp.float32),
            jax.ShapeDtypeStruct((_LAYERS, 2, _N, 128), jnp.float32),
            jax.ShapeDtypeStruct((_N, 2 * _LAYERS), jnp.float32),
        ],
    )(x, lin1_W, lin1_b.reshape(1, _DIM), gat_W, att_src, att_dst)


def _kernel_c(x0_ref, og_ref, den_ref, gb_ref, wih_ref, whh_ref,
              w2_ref, b2_ref, out_ref):
    xc = x0_ref[...]
    h = jnp.zeros((_BN, _HID), dtype=jnp.float32)
    c = jnp.zeros((_BN, _HID), dtype=jnp.float32)
    for l in range(_LAYERS):
        og = jnp.concatenate([og_ref[l, 0], og_ref[l, 1]], axis=1)
        den = den_ref[:, l:l + 1] + 1e-16
        ht = jnp.tanh(og / den + gb_ref[l][None, :])
        incat = jnp.concatenate([ht, xc], axis=1)
        gates = (jnp.dot(incat, wih_ref[l], preferred_element_type=jnp.float32)
                 + jnp.dot(h, whh_ref[l], preferred_element_type=jnp.float32))
        gi = jax.nn.sigmoid(gates[:, 0:_HID])
        gf = jax.nn.sigmoid(gates[:, _HID:2 * _HID])
        gg = jnp.tanh(gates[:, 2 * _HID:3 * _HID])
        go = jax.nn.sigmoid(gates[:, 3 * _HID:4 * _HID])
        c = gf * c + gi * gg
        h = go * jnp.tanh(c)
        xc = h
    out_ref[...] = jnp.dot(xc, w2_ref[...],
                           preferred_element_type=jnp.float32) + b2_ref[...]


def _run_c(x0, og, den, gat_b, wihT, whhT, lin2_W, lin2_b):
    nblk = _N // _BN
    full = lambda *shape: pl.BlockSpec(shape, lambda i: (0,) * len(shape))
    return pl.pallas_call(
        _kernel_c,
        grid=(nblk,),
        in_specs=[
            pl.BlockSpec((_BN, _DIM), lambda i: (i, 0)),
            pl.BlockSpec((_LAYERS, 2, _BN, 128), lambda i: (0, 0, i, 0)),
            pl.BlockSpec((_BN, _LAYERS), lambda i: (i, 0)),
            full(_LAYERS, _DIM),
            full(_LAYERS, 2 * _DIM, 4 * _HID),
            full(_LAYERS, _HID, 4 * _HID),
            full(_DIM, _OUT_DIM),
            full(1, _OUT_DIM),
        ],
        out_specs=pl.BlockSpec((_BN, _OUT_DIM), lambda i: (i, 0)),
        out_shape=jax.ShapeDtypeStruct((_N, _OUT_DIM), jnp.float32),
    )(x0, og, den, gat_b, wihT, whhT, lin2_W, lin2_b.reshape(1, _OUT_DIM))


_CK = 96
_NCHUNK = 216
_EP_TILE = _NCHUNK * _CK
_EPAD = 16 * _EP_TILE
_E1 = _E + _N
_NROW = 624
_NTAIL = _N - 16 * _NROW


def _sc_edges(src, dst, aa_s, aa_d, h, z_rows):
    mesh = plsc.VectorSubcoreMesh(core_axis_name="c", subcore_axis_name="s")

    @functools.partial(
        pl.kernel,
        mesh=mesh,
        out_type=[
            jax.ShapeDtypeStruct((_LAYERS, 2, _N, 128), jnp.float32),
            jax.ShapeDtypeStruct((_LAYERS * _N,), jnp.float32),
        ],
        scratch_types=[
            pltpu.VMEM_SHARED((_N, 128), jnp.float32),
            pltpu.VMEM_SHARED((_N,), jnp.float32),
            pltpu.VMEM((_N,), jnp.float32),
            pltpu.VMEM((_N,), jnp.float32),
            pltpu.VMEM((_CK,), jnp.int32),
            pltpu.VMEM((_CK,), jnp.int32),
            pltpu.VMEM((_CK,), jnp.int32),
            pltpu.VMEM((_CK,), jnp.int32),
            pltpu.VMEM((_CK,), jnp.int32),
            pltpu.VMEM((_CK, 128), jnp.float32),
            pltpu.VMEM((_CK, 128), jnp.float32),
            pltpu.VMEM((_CK,), jnp.float32),
            pltpu.VMEM((_CK,), jnp.float32),
            pltpu.VMEM((_NROW,), jnp.float32),
            pltpu.SemaphoreType.DMA,
            pltpu.SemaphoreType.DMA,
            pltpu.SemaphoreType.DMA,
        ],
        compiler_params=pltpu.CompilerParams(needs_layout_passes=False),
    )
    def k(src_h, dst_h, aas_h, aad_h, h_h, zr_h, og_h, den_h,
          sp_out, sp_den, v_as, v_ad, v_src0, v_src1, v_dst0, v_dst1,
          v_dst2, v_rows0, v_rows1, v_e0, v_e1, v_den, sem_g, sem_i, sem_s):
        cid = lax.axis_index("c")
        sid = lax.axis_index("s")
        rbase = sid * _NROW
        ebase = sid * _EP_TILE
        rows = (v_rows0, v_rows1)
        sbuf = (v_src0, v_src1)
        dbuf = (v_dst0, v_dst1, v_dst2)
        ebuf = (v_e0, v_e1)

        def idx_issue(c, b, b3):
            off = ebase + c * _CK
            pltpu.async_copy(src_h.at[pl.ds(off, _CK)], sbuf[b], sem_i)
            pltpu.async_copy(dst_h.at[pl.ds(off, _CK)], dbuf[b3], sem_i)

        def idx_drain(c, b, b3):
            off = ebase + c * _CK
            pltpu.make_async_copy(src_h.at[pl.ds(off, _CK)], sbuf[b],
                                  sem_i).wait()
            pltpu.make_async_copy(dst_h.at[pl.ds(off, _CK)], dbuf[b3],
                                  sem_i).wait()

        def sc_issue(b, b3):
            pltpu.async_copy(rows[b], sp_out.at[dbuf[b3]], sem_s, add=True)
            pltpu.async_copy(ebuf[b], sp_den.at[dbuf[b3]], sem_s, add=True)

        def sc_drain(b, b3):
            pltpu.make_async_copy(rows[b], sp_out.at[dbuf[b3]],
                                  sem_s).wait()
            pltpu.make_async_copy(ebuf[b], sp_den.at[dbuf[b3]],
                                  sem_s).wait()

        def gat_issue(l, b):
            @pl.when(cid == 0)
            def _():
                pltpu.async_copy(h_h.at[l].at[0].at[sbuf[b]], rows[b], sem_g)

            @pl.when(cid == 1)
            def _():
                pltpu.async_copy(h_h.at[l].at[1].at[sbuf[b]], rows[b], sem_g)

        def gat_drain(l, b):
            pltpu.make_async_copy(h_h.at[l].at[0].at[sbuf[b]], rows[b],
                                  sem_g).wait()

        def layer(l, carry):
            pltpu.sync_copy(aas_h.at[pl.ds(l * _N, _N)], v_as)
            pltpu.sync_copy(aad_h.at[pl.ds(l * _N, _N)], v_ad)
            pltpu.sync_copy(zr_h, v_rows0)
            for p in range(6):
                pltpu.sync_copy(v_rows0,
                                sp_out.at[pl.ds(rbase + p * 96, 96)])
            pltpu.sync_copy(v_rows0.at[pl.ds(0, 48)],
                            sp_out.at[pl.ds(rbase + 576, 48)])
            for p in range(4):
                pltpu.sync_copy(v_rows0.at[p],
                                sp_den.at[pl.ds(rbase + p * 128, 128)])
            pltpu.sync_copy(v_rows0.at[4, pl.ds(0, 112)],
                            sp_den.at[pl.ds(rbase + 512, 112)])

            @pl.when(sid == 0)
            def _():
                pltpu.sync_copy(v_rows0.at[pl.ds(0, _NTAIL)],
                                sp_out.at[pl.ds(16 * _NROW, _NTAIL)])
                pltpu.sync_copy(v_rows0.at[5, pl.ds(0, _NTAIL)],
                                sp_den.at[pl.ds(16 * _NROW, _NTAIL)])

            plsc.subcore_barrier()

            idx_issue(0, 0, 0)
            idx_drain(0, 0, 0)
            idx_issue(1, 1, 1)
            gat_issue(l, 0)

            def six(kk, bcarry):
                for b6 in range(6):
                    c = kk * 6 + b6
                    b = b6 % 2
                    b3 = b6 % 3
                    off = ebase + c * _CK
                    for g in range(_CK // 16):
                        sidx = sbuf[b][pl.ds(g * 16, 16)]
                        didx = dbuf[b3][pl.ds(g * 16, 16)]
                        a = (plsc.load_gather(v_as, [sidx])
                             + plsc.load_gather(v_ad, [didx]))
                        a = jnp.where(a > 0, a, 0.2 * a)
                        e = jnp.exp(a)
                        gi = off + g * 16 + lax.iota(jnp.int32, 16)
                        ebuf[b][pl.ds(g * 16, 16)] = jnp.where(
                            gi < _E1, e, 0.0)

                    @pl.when(c + 1 < _NCHUNK)
                    def _():
                        idx_drain(c + 1, 1 - b, (b3 + 1) % 3)

                    @pl.when(c >= 1)
                    def _():
                        sc_drain(1 - b, (b3 + 2) % 3)

                    @pl.when(c + 1 < _NCHUNK)
                    def _():
                        gat_issue(l, 1 - b)
                    gat_drain(l, b)

                    @plsc.parallel_loop(0, _CK // 16)
                    def _(gg):
                        e_vec = ebuf[b][pl.ds(gg * 16, 16)]
                        for t in range(16):
                            eb = jnp.broadcast_to(e_vec[t], (16,))
                            jr = gg * 16 + t
                            for q in range(8):
                                rows[b][jr, pl.ds(q * 16, 16)] = (
                                    rows[b][jr, pl.ds(q * 16, 16)] * eb)
                    sc_issue(b, b3)

                    @pl.when(c + 2 < _NCHUNK)
                    def _():
                        idx_issue(c + 2, b, (b3 + 2) % 3)
                return bcarry
            lax.fori_loop(0, _NCHUNK // 6, six, 0)
            sc_drain(1, 2)
            plsc.subcore_barrier()

            @pl.when(cid == 0)
            def _():
                pltpu.sync_copy(sp_out.at[pl.ds(rbase, _NROW)],
                                og_h.at[l, 0, pl.ds(rbase, _NROW)])
                pltpu.sync_copy(sp_den.at[pl.ds(rbase, _NROW)], v_den)
                pltpu.sync_copy(v_den, den_h.at[pl.ds(l * _N + rbase, _NROW)])

            @pl.when(cid == 1)
            def _():
                pltpu.sync_copy(sp_out.at[pl.ds(rbase, _NROW)],
                                og_h.at[l, 1, pl.ds(rbase, _NROW)])

            @pl.when((sid == 0) & (cid == 0))
            def _():
                pltpu.sync_copy(sp_out.at[pl.ds(16 * _NROW, _NTAIL)],
                                og_h.at[l, 0, pl.ds(16 * _NROW, _NTAIL)])
                pltpu.sync_copy(sp_den.at[pl.ds(16 * _NROW, _NTAIL)],
                                v_den.at[pl.ds(0, _NTAIL)])
                pltpu.sync_copy(v_den.at[pl.ds(0, _NTAIL)],
                                den_h.at[pl.ds(l * _N + 16 * _NROW, _NTAIL)])

            @pl.when((sid == 0) & (cid == 1))
            def _():
                pltpu.sync_copy(sp_out.at[pl.ds(16 * _NROW, _NTAIL)],
                                og_h.at[l, 1, pl.ds(16 * _NROW, _NTAIL)])

            plsc.subcore_barrier()
            return carry
        lax.fori_loop(0, _LAYERS, layer, 0)

    return k(src, dst, aa_s, aa_d, h, z_rows)


def kernel(x, edge_index, lin1_W, lin1_b, gat_W, att_src, att_dst, gat_b,
           lstm_Wih, lstm_Whh, lin2_W, lin2_b):
    loop = jnp.arange(_N, dtype=edge_index.dtype)
    pad = jnp.zeros((_EPAD - _E1,), dtype=edge_index.dtype)
    src = jnp.concatenate([edge_index[0], loop, pad])
    dst = jnp.concatenate([edge_index[1], loop, pad])
    x0, h, aa = _run_a(x, lin1_W, lin1_b, gat_W, att_src, att_dst)
    aa_s = jnp.transpose(aa[:, 0::2])
    aa_d = jnp.transpose(aa[:, 1::2])
    z_rows = jnp.zeros((_CK, 128), jnp.float32)
    og, den = _sc_edges(src, dst, aa_s.reshape(-1), aa_d.reshape(-1), h,
                        z_rows)
    den = den.reshape(_LAYERS, _N)
    wihT = jnp.transpose(lstm_Wih, (0, 2, 1))
    whhT = jnp.transpose(lstm_Whh, (0, 2, 1))
    return _run_c(x0, og, jnp.transpose(den), gat_b, wihT, whhT, lin2_W, lin2_b)


# speedup vs baseline: 1.0299x; 1.0299x over previous
import functools

import jax
import jax.numpy as jnp
from jax import lax
from jax.experimental import pallas as pl
from jax.experimental.pallas import tpu as pltpu
from jax.experimental.pallas import tpu_sc as plsc

_N = 10000
_E = 320000
_IN_DIM = 128
_OUT_DIM = 128
_DIM = 256
_HID = 256
_LAYERS = 4

_BN = 1000


def _kernel_a(x_ref, w1_ref, b1_ref, wg_ref, asrc_ref, adst_ref,
              x0_ref, h_ref, aa_ref):
    x0 = jnp.dot(x_ref[...], w1_ref[...],
                 preferred_element_type=jnp.float32) + b1_ref[...]
    x0_ref[...] = x0
    cols = []
    for l in range(_LAYERS):
        h = jnp.dot(x0, wg_ref[l], preferred_element_type=jnp.float32)
        h_ref[l, 0] = h[:, :128]
        h_ref[l, 1] = h[:, 128:]
        cols.append(jnp.sum(h * asrc_ref[l][None, :], axis=1))
        cols.append(jnp.sum(h * adst_ref[l][None, :], axis=1))
    aa_ref[...] = jnp.stack(cols, axis=1)


def _run_a(x, lin1_W, lin1_b, gat_W, att_src, att_dst):
    nblk = _N // _BN
    full = lambda *shape: pl.BlockSpec(shape, lambda i: (0,) * len(shape))
    return pl.pallas_call(
        _kernel_a,
        grid=(nblk,),
        in_specs=[
            pl.BlockSpec((_BN, _IN_DIM), lambda i: (i, 0)),
            full(_IN_DIM, _DIM),
            full(1, _DIM),
            full(_LAYERS, _DIM, _DIM),
            full(_LAYERS, _DIM),
            full(_LAYERS, _DIM),
        ],
        out_specs=[
            pl.BlockSpec((_BN, _DIM), lambda i: (i, 0)),
            pl.BlockSpec((_LAYERS, 2, _BN, 128), lambda i: (0, 0, i, 0)),
            pl.BlockSpec((_BN, 2 * _LAYERS), lambda i: (i, 0)),
        ],
        out_shape=[
            jax.ShapeDtypeStruct((_N, _DIM), jnp.float32),
            jax.ShapeDtypeStruct((_LAYERS, 2, _N, 128), jnp.float32),
            jax.ShapeDtypeStruct((_N, 2 * _LAYERS), jnp.float32),
        ],
    )(x, lin1_W, lin1_b.reshape(1, _DIM), gat_W, att_src, att_dst)


def _kernel_c(x0_ref, og_ref, den_ref, gb_ref, wih_ref, whh_ref,
              w2_ref, b2_ref, out_ref):
    xc = x0_ref[...]
    h = jnp.zeros((_BN, _HID), dtype=jnp.float32)
    c = jnp.zeros((_BN, _HID), dtype=jnp.float32)
    for l in range(_LAYERS):
        og = jnp.concatenate([og_ref[l, 0], og_ref[l, 1]], axis=1)
        den = den_ref[:, l:l + 1] + 1e-16
        ht = jnp.tanh(og / den + gb_ref[l][None, :])
        incat = jnp.concatenate([ht, xc], axis=1)
        gates = (jnp.dot(incat, wih_ref[l], preferred_element_type=jnp.float32)
                 + jnp.dot(h, whh_ref[l], preferred_element_type=jnp.float32))
        gi = jax.nn.sigmoid(gates[:, 0:_HID])
        gf = jax.nn.sigmoid(gates[:, _HID:2 * _HID])
        gg = jnp.tanh(gates[:, 2 * _HID:3 * _HID])
        go = jax.nn.sigmoid(gates[:, 3 * _HID:4 * _HID])
        c = gf * c + gi * gg
        h = go * jnp.tanh(c)
        xc = h
    out_ref[...] = jnp.dot(xc, w2_ref[...],
                           preferred_element_type=jnp.float32) + b2_ref[...]


def _run_c(x0, og, den, gat_b, wihT, whhT, lin2_W, lin2_b):
    nblk = _N // _BN
    full = lambda *shape: pl.BlockSpec(shape, lambda i: (0,) * len(shape))
    return pl.pallas_call(
        _kernel_c,
        grid=(nblk,),
        in_specs=[
            pl.BlockSpec((_BN, _DIM), lambda i: (i, 0)),
            pl.BlockSpec((_LAYERS, 2, _BN, 128), lambda i: (0, 0, i, 0)),
            pl.BlockSpec((_BN, _LAYERS), lambda i: (i, 0)),
            full(_LAYERS, _DIM),
            full(_LAYERS, 2 * _DIM, 4 * _HID),
            full(_LAYERS, _HID, 4 * _HID),
            full(_DIM, _OUT_DIM),
            full(1, _OUT_DIM),
        ],
        out_specs=pl.BlockSpec((_BN, _OUT_DIM), lambda i: (i, 0)),
        out_shape=jax.ShapeDtypeStruct((_N, _OUT_DIM), jnp.float32),
    )(x0, og, den, gat_b, wihT, whhT, lin2_W, lin2_b.reshape(1, _OUT_DIM))


_CK = 96
_NCHUNK = 216
_EP_TILE = _NCHUNK * _CK
_EPAD = 16 * _EP_TILE
_E1 = _E + _N
_NROW = 624
_NTAIL = _N - 16 * _NROW


def _sc_edges(src, dst, aa_s, aa_d, h, z_rows):
    mesh = plsc.VectorSubcoreMesh(core_axis_name="c", subcore_axis_name="s")

    @functools.partial(
        pl.kernel,
        mesh=mesh,
        out_type=[
            jax.ShapeDtypeStruct((_LAYERS, 2, _N, 128), jnp.float32),
            jax.ShapeDtypeStruct((_LAYERS * _N,), jnp.float32),
        ],
        scratch_types=[
            pltpu.VMEM_SHARED((_N, 128), jnp.float32),
            pltpu.VMEM_SHARED((_N,), jnp.float32),
            pltpu.VMEM((_N,), jnp.float32),
            pltpu.VMEM((_N,), jnp.float32),
            pltpu.VMEM((_CK,), jnp.int32),
            pltpu.VMEM((_CK,), jnp.int32),
            pltpu.VMEM((_CK,), jnp.int32),
            pltpu.VMEM((_CK,), jnp.int32),
            pltpu.VMEM((_CK,), jnp.int32),
            pltpu.VMEM((_CK, 128), jnp.float32),
            pltpu.VMEM((_CK, 128), jnp.float32),
            pltpu.VMEM((_CK,), jnp.float32),
            pltpu.VMEM((_CK,), jnp.float32),
            pltpu.VMEM((_NROW,), jnp.float32),
            pltpu.SemaphoreType.DMA,
            pltpu.SemaphoreType.DMA,
            pltpu.SemaphoreType.DMA,
        ],
        compiler_params=pltpu.CompilerParams(needs_layout_passes=False),
    )
    def k(src_h, dst_h, aas_h, aad_h, h_h, zr_h, og_h, den_h,
          sp_out, sp_den, v_as, v_ad, v_src0, v_src1, v_dst0, v_dst1,
          v_dst2, v_rows0, v_rows1, v_e0, v_e1, v_den, sem_g, sem_i, sem_s):
        cid = lax.axis_index("c")
        sid = lax.axis_index("s")
        rbase = sid * _NROW
        ebase = sid * _EP_TILE
        rows = (v_rows0, v_rows1)
        sbuf = (v_src0, v_src1)
        dbuf = (v_dst0, v_dst1, v_dst2)
        ebuf = (v_e0, v_e1)

        def idx_issue(c, b, b3):
            off = ebase + c * _CK
            pltpu.async_copy(src_h.at[pl.ds(off, _CK)], sbuf[b], sem_i)
            pltpu.async_copy(dst_h.at[pl.ds(off, _CK)], dbuf[b3], sem_i)

        def idx_drain(c, b, b3):
            off = ebase + c * _CK
            pltpu.make_async_copy(src_h.at[pl.ds(off, _CK)], sbuf[b],
                                  sem_i).wait()
            pltpu.make_async_copy(dst_h.at[pl.ds(off, _CK)], dbuf[b3],
                                  sem_i).wait()

        def sc_issue(b, b3):
            pltpu.async_copy(rows[b], sp_out.at[dbuf[b3]], sem_s, add=True)
            pltpu.async_copy(ebuf[b], sp_den.at[dbuf[b3]], sem_s, add=True)

        def sc_drain(b, b3):
            pltpu.make_async_copy(rows[b], sp_out.at[dbuf[b3]],
                                  sem_s).wait()
            pltpu.make_async_copy(ebuf[b], sp_den.at[dbuf[b3]],
                                  sem_s).wait()

        def gat_issue(l, b):
            @pl.when(cid == 0)
            def _():
                pltpu.async_copy(h_h.at[l].at[0].at[sbuf[b]], rows[b], sem_g)

            @pl.when(cid == 1)
            def _():
                pltpu.async_copy(h_h.at[l].at[1].at[sbuf[b]], rows[b], sem_g)

        def gat_drain(l, b):
            pltpu.make_async_copy(h_h.at[l].at[0].at[sbuf[b]], rows[b],
                                  sem_g).wait()

        def layer(l, carry):
            pltpu.sync_copy(aas_h.at[pl.ds(l * _N, _N)], v_as)
            pltpu.sync_copy(aad_h.at[pl.ds(l * _N, _N)], v_ad)
            pltpu.sync_copy(zr_h, v_rows0)
            for p in range(6):
                pltpu.sync_copy(v_rows0,
                                sp_out.at[pl.ds(rbase + p * 96, 96)])
            pltpu.sync_copy(v_rows0.at[pl.ds(0, 48)],
                            sp_out.at[pl.ds(rbase + 576, 48)])
            for p in range(4):
                pltpu.sync_copy(v_rows0.at[p],
                                sp_den.at[pl.ds(rbase + p * 128, 128)])
            pltpu.sync_copy(v_rows0.at[4, pl.ds(0, 112)],
                            sp_den.at[pl.ds(rbase + 512, 112)])

            @pl.when(sid == 0)
            def _():
                pltpu.sync_copy(v_rows0.at[pl.ds(0, _NTAIL)],
                                sp_out.at[pl.ds(16 * _NROW, _NTAIL)])
                pltpu.sync_copy(v_rows0.at[5, pl.ds(0, _NTAIL)],
                                sp_den.at[pl.ds(16 * _NROW, _NTAIL)])

            plsc.subcore_barrier()

            idx_issue(0, 0, 0)
            idx_drain(0, 0, 0)
            idx_issue(1, 1, 1)
            gat_issue(l, 0)

            def six(kk, bcarry):
                for b6 in range(6):
                    c = kk * 6 + b6
                    b = b6 % 2
                    b3 = b6 % 3
                    off = ebase + c * _CK
                    for g in range(_CK // 16):
                        sidx = sbuf[b][pl.ds(g * 16, 16)]
                        didx = dbuf[b3][pl.ds(g * 16, 16)]
                        a = (plsc.load_gather(v_as, [sidx])
                             + plsc.load_gather(v_ad, [didx]))
                        a = jnp.where(a > 0, a, 0.2 * a)
                        e = jnp.exp(a)
                        gi = off + g * 16 + lax.iota(jnp.int32, 16)
                        ebuf[b][pl.ds(g * 16, 16)] = jnp.where(
                            gi < _E1, e, 0.0)

                    @pl.when(c + 1 < _NCHUNK)
                    def _():
                        idx_drain(c + 1, 1 - b, (b3 + 1) % 3)

                    @pl.when(c >= 1)
                    def _():
                        sc_drain(1 - b, (b3 + 2) % 3)

                    @pl.when(c + 1 < _NCHUNK)
                    def _():
                        gat_issue(l, 1 - b)
                    gat_drain(l, b)

                    @plsc.parallel_loop(0, _CK, unroll=16)
                    def _(jr):
                        eb = plsc.load_gather(
                            ebuf[b], [jnp.full((16,), jr, jnp.int32)])
                        for q in range(8):
                            rows[b][jr, pl.ds(q * 16, 16)] = (
                                rows[b][jr, pl.ds(q * 16, 16)] * eb)
                    sc_issue(b, b3)

                    @pl.when(c + 2 < _NCHUNK)
                    def _():
                        idx_issue(c + 2, b, (b3 + 2) % 3)
                return bcarry
            lax.fori_loop(0, _NCHUNK // 6, six, 0)
            sc_drain(1, 2)
            plsc.subcore_barrier()

            @pl.when(cid == 0)
            def _():
                pltpu.sync_copy(sp_out.at[pl.ds(rbase, _NROW)],
                                og_h.at[l, 0, pl.ds(rbase, _NROW)])
                pltpu.sync_copy(sp_den.at[pl.ds(rbase, _NROW)], v_den)
                pltpu.sync_copy(v_den, den_h.at[pl.ds(l * _N + rbase, _NROW)])

            @pl.when(cid == 1)
            def _():
                pltpu.sync_copy(sp_out.at[pl.ds(rbase, _NROW)],
                                og_h.at[l, 1, pl.ds(rbase, _NROW)])

            @pl.when((sid == 0) & (cid == 0))
            def _():
                pltpu.sync_copy(sp_out.at[pl.ds(16 * _NROW, _NTAIL)],
                                og_h.at[l, 0, pl.ds(16 * _NROW, _NTAIL)])
                pltpu.sync_copy(sp_den.at[pl.ds(16 * _NROW, _NTAIL)],
                                v_den.at[pl.ds(0, _NTAIL)])
                pltpu.sync_copy(v_den.at[pl.ds(0, _NTAIL)],
                                den_h.at[pl.ds(l * _N + 16 * _NROW, _NTAIL)])

            @pl.when((sid == 0) & (cid == 1))
            def _():
                pltpu.sync_copy(sp_out.at[pl.ds(16 * _NROW, _NTAIL)],
                                og_h.at[l, 1, pl.ds(16 * _NROW, _NTAIL)])

            plsc.subcore_barrier()
            return carry
        lax.fori_loop(0, _LAYERS, layer, 0)

    return k(src, dst, aa_s, aa_d, h, z_rows)


def kernel(x, edge_index, lin1_W, lin1_b, gat_W, att_src, att_dst, gat_b,
           lstm_Wih, lstm_Whh, lin2_W, lin2_b):
    loop = jnp.arange(_N, dtype=edge_index.dtype)
    pad = jnp.zeros((_EPAD - _E1,), dtype=edge_index.dtype)
    src = jnp.concatenate([edge_index[0], loop, pad])
    dst = jnp.concatenate([edge_index[1], loop, pad])
    x0, h, aa = _run_a(x, lin1_W, lin1_b, gat_W, att_src, att_dst)
    aa_s = jnp.transpose(aa[:, 0::2])
    aa_d = jnp.transpose(aa[:, 1::2])
    z_rows = jnp.zeros((_CK, 128), jnp.float32)
    og, den = _sc_edges(src, dst, aa_s.reshape(-1), aa_d.reshape(-1), h,
                        z_rows)
    den = den.reshape(_LAYERS, _N)
    wihT = jnp.transpose(lstm_Wih, (0, 2, 1))
    whhT = jnp.transpose(lstm_Whh, (0, 2, 1))
    return _run_c(x0, og, jnp.transpose(den), gat_b, wihT, whhT, lin2_W, lin2_b)


# speedup vs baseline: 1.0650x; 1.0341x over previous
import functools

import jax
import jax.numpy as jnp
from jax import lax
from jax.experimental import pallas as pl
from jax.experimental.pallas import tpu as pltpu
from jax.experimental.pallas import tpu_sc as plsc

_N = 10000
_E = 320000
_IN_DIM = 128
_OUT_DIM = 128
_DIM = 256
_HID = 256
_LAYERS = 4

_BN = 1000


def _kernel_a(x_ref, w1_ref, b1_ref, wg_ref, asrc_ref, adst_ref,
              x0_ref, h_ref, aa_ref):
    x0 = jnp.dot(x_ref[...], w1_ref[...],
                 preferred_element_type=jnp.float32) + b1_ref[...]
    x0_ref[...] = x0
    cols = []
    for l in range(_LAYERS):
        h = jnp.dot(x0, wg_ref[l], preferred_element_type=jnp.float32)
        h_ref[l, 0] = h[:, :128]
        h_ref[l, 1] = h[:, 128:]
        cols.append(jnp.sum(h * asrc_ref[l][None, :], axis=1))
        cols.append(jnp.sum(h * adst_ref[l][None, :], axis=1))
    aa_ref[...] = jnp.stack(cols, axis=1)


def _run_a(x, lin1_W, lin1_b, gat_W, att_src, att_dst):
    nblk = _N // _BN
    full = lambda *shape: pl.BlockSpec(shape, lambda i: (0,) * len(shape))
    return pl.pallas_call(
        _kernel_a,
        grid=(nblk,),
        in_specs=[
            pl.BlockSpec((_BN, _IN_DIM), lambda i: (i, 0)),
            full(_IN_DIM, _DIM),
            full(1, _DIM),
            full(_LAYERS, _DIM, _DIM),
            full(_LAYERS, _DIM),
            full(_LAYERS, _DIM),
        ],
        out_specs=[
            pl.BlockSpec((_BN, _DIM), lambda i: (i, 0)),
            pl.BlockSpec((_LAYERS, 2, _BN, 128), lambda i: (0, 0, i, 0)),
            pl.BlockSpec((_BN, 2 * _LAYERS), lambda i: (i, 0)),
        ],
        out_shape=[
            jax.ShapeDtypeStruct((_N, _DIM), jnp.float32),
            jax.ShapeDtypeStruct((_LAYERS, 2, _N, 128), jnp.float32),
            jax.ShapeDtypeStruct((_N, 2 * _LAYERS), jnp.float32),
        ],
    )(x, lin1_W, lin1_b.reshape(1, _DIM), gat_W, att_src, att_dst)


def _kernel_c(x0_ref, og_ref, den_ref, gb_ref, wih_ref, whh_ref,
              w2_ref, b2_ref, out_ref):
    xc = x0_ref[...]
    h = jnp.zeros((_BN, _HID), dtype=jnp.float32)
    c = jnp.zeros((_BN, _HID), dtype=jnp.float32)
    for l in range(_LAYERS):
        og = jnp.concatenate([og_ref[l, 0], og_ref[l, 1]], axis=1)
        den = den_ref[:, l:l + 1] + 1e-16
        ht = jnp.tanh(og / den + gb_ref[l][None, :])
        incat = jnp.concatenate([ht, xc], axis=1)
        gates = (jnp.dot(incat, wih_ref[l], preferred_element_type=jnp.float32)
                 + jnp.dot(h, whh_ref[l], preferred_element_type=jnp.float32))
        gi = jax.nn.sigmoid(gates[:, 0:_HID])
        gf = jax.nn.sigmoid(gates[:, _HID:2 * _HID])
        gg = jnp.tanh(gates[:, 2 * _HID:3 * _HID])
        go = jax.nn.sigmoid(gates[:, 3 * _HID:4 * _HID])
        c = gf * c + gi * gg
        h = go * jnp.tanh(c)
        xc = h
    out_ref[...] = jnp.dot(xc, w2_ref[...],
                           preferred_element_type=jnp.float32) + b2_ref[...]


def _run_c(x0, og, den, gat_b, wihT, whhT, lin2_W, lin2_b):
    nblk = _N // _BN
    full = lambda *shape: pl.BlockSpec(shape, lambda i: (0,) * len(shape))
    return pl.pallas_call(
        _kernel_c,
        grid=(nblk,),
        in_specs=[
            pl.BlockSpec((_BN, _DIM), lambda i: (i, 0)),
            pl.BlockSpec((_LAYERS, 2, _BN, 128), lambda i: (0, 0, i, 0)),
            pl.BlockSpec((_BN, _LAYERS), lambda i: (i, 0)),
            full(_LAYERS, _DIM),
            full(_LAYERS, 2 * _DIM, 4 * _HID),
            full(_LAYERS, _HID, 4 * _HID),
            full(_DIM, _OUT_DIM),
            full(1, _OUT_DIM),
        ],
        out_specs=pl.BlockSpec((_BN, _OUT_DIM), lambda i: (i, 0)),
        out_shape=jax.ShapeDtypeStruct((_N, _OUT_DIM), jnp.float32),
    )(x0, og, den, gat_b, wihT, whhT, lin2_W, lin2_b.reshape(1, _OUT_DIM))


_CK = 96
_NCHUNK = 216
_EP_TILE = _NCHUNK * _CK
_EPAD = 16 * _EP_TILE
_E1 = _E + _N
_NROW = 624
_NTAIL = _N - 16 * _NROW


def _sc_edges(src, dst, aa_s, aa_d, h, z_rows):
    mesh = plsc.VectorSubcoreMesh(core_axis_name="c", subcore_axis_name="s")

    @functools.partial(
        pl.kernel,
        mesh=mesh,
        out_type=[
            jax.ShapeDtypeStruct((_LAYERS, 2, _N, 128), jnp.float32),
            jax.ShapeDtypeStruct((_LAYERS * _N,), jnp.float32),
        ],
        scratch_types=[
            pltpu.VMEM_SHARED((_N, 128), jnp.float32),
            pltpu.VMEM_SHARED((_N,), jnp.float32),
            pltpu.VMEM((_N,), jnp.float32),
            pltpu.VMEM((_N,), jnp.float32),
            pltpu.VMEM((_CK,), jnp.int32),
            pltpu.VMEM((_CK,), jnp.int32),
            pltpu.VMEM((_CK,), jnp.int32),
            pltpu.VMEM((_CK,), jnp.int32),
            pltpu.VMEM((_CK,), jnp.int32),
            pltpu.VMEM((_CK, 128), jnp.float32),
            pltpu.VMEM((_CK, 128), jnp.float32),
            pltpu.VMEM((_CK,), jnp.float32),
            pltpu.VMEM((_CK,), jnp.float32),
            pltpu.VMEM((_NROW,), jnp.float32),
            pltpu.SemaphoreType.DMA,
            pltpu.SemaphoreType.DMA,
            pltpu.SemaphoreType.DMA,
        ],
        compiler_params=pltpu.CompilerParams(needs_layout_passes=False),
    )
    def k(src_h, dst_h, aas_h, aad_h, h_h, zr_h, og_h, den_h,
          sp_out, sp_den, v_as, v_ad, v_src0, v_src1, v_dst0, v_dst1,
          v_dst2, v_rows0, v_rows1, v_e0, v_e1, v_den, sem_g, sem_i, sem_s):
        cid = lax.axis_index("c")
        sid = lax.axis_index("s")
        rbase = sid * _NROW
        ebase = sid * _EP_TILE
        rows = (v_rows0, v_rows1)
        sbuf = (v_src0, v_src1)
        dbuf = (v_dst0, v_dst1, v_dst2)
        ebuf = (v_e0, v_e1)

        def idx_issue(c, b, b3):
            off = ebase + c * _CK
            pltpu.async_copy(src_h.at[pl.ds(off, _CK)], sbuf[b], sem_i)
            pltpu.async_copy(dst_h.at[pl.ds(off, _CK)], dbuf[b3], sem_i)

        def idx_drain(c, b, b3):
            off = ebase + c * _CK
            pltpu.make_async_copy(src_h.at[pl.ds(off, _CK)], sbuf[b],
                                  sem_i).wait()
            pltpu.make_async_copy(dst_h.at[pl.ds(off, _CK)], dbuf[b3],
                                  sem_i).wait()

        def sc_issue(b, b3):
            pltpu.async_copy(rows[b], sp_out.at[dbuf[b3]], sem_s, add=True)
            pltpu.async_copy(ebuf[b], sp_den.at[dbuf[b3]], sem_s, add=True)

        def sc_drain(b, b3):
            pltpu.make_async_copy(rows[b], sp_out.at[dbuf[b3]],
                                  sem_s).wait()
            pltpu.make_async_copy(ebuf[b], sp_den.at[dbuf[b3]],
                                  sem_s).wait()

        def gat_issue(l, b):
            @pl.when(cid == 0)
            def _():
                pltpu.async_copy(h_h.at[l].at[0].at[sbuf[b]], rows[b], sem_g)

            @pl.when(cid == 1)
            def _():
                pltpu.async_copy(h_h.at[l].at[1].at[sbuf[b]], rows[b], sem_g)

        def gat_drain(l, b):
            pltpu.make_async_copy(h_h.at[l].at[0].at[sbuf[b]], rows[b],
                                  sem_g).wait()

        def layer(l, carry):
            pltpu.sync_copy(aas_h.at[pl.ds(l * _N, _N)], v_as)
            pltpu.sync_copy(aad_h.at[pl.ds(l * _N, _N)], v_ad)
            pltpu.sync_copy(zr_h, v_rows0)
            for p in range(6):
                pltpu.sync_copy(v_rows0,
                                sp_out.at[pl.ds(rbase + p * 96, 96)])
            pltpu.sync_copy(v_rows0.at[pl.ds(0, 48)],
                            sp_out.at[pl.ds(rbase + 576, 48)])
            for p in range(4):
                pltpu.sync_copy(v_rows0.at[p],
                                sp_den.at[pl.ds(rbase + p * 128, 128)])
            pltpu.sync_copy(v_rows0.at[4, pl.ds(0, 112)],
                            sp_den.at[pl.ds(rbase + 512, 112)])

            @pl.when(sid == 0)
            def _():
                pltpu.sync_copy(v_rows0.at[pl.ds(0, _NTAIL)],
                                sp_out.at[pl.ds(16 * _NROW, _NTAIL)])
                pltpu.sync_copy(v_rows0.at[5, pl.ds(0, _NTAIL)],
                                sp_den.at[pl.ds(16 * _NROW, _NTAIL)])

            plsc.subcore_barrier()

            idx_issue(0, 0, 0)
            idx_drain(0, 0, 0)
            idx_issue(1, 1, 1)
            gat_issue(l, 0)

            def six(kk, bcarry):
                for b6 in range(6):
                    c = kk * 6 + b6
                    b = b6 % 2
                    b3 = b6 % 3
                    off = ebase + c * _CK
                    for g in range(_CK // 16):
                        sidx = sbuf[b][pl.ds(g * 16, 16)]
                        didx = dbuf[b3][pl.ds(g * 16, 16)]
                        a = (plsc.load_gather(v_as, [sidx])
                             + plsc.load_gather(v_ad, [didx]))
                        a = jnp.where(a > 0, a, 0.2 * a)
                        e = jnp.exp(a)
                        gi = off + g * 16 + lax.iota(jnp.int32, 16)
                        ebuf[b][pl.ds(g * 16, 16)] = jnp.where(
                            gi < _E1, e, 0.0)

                    @pl.when(c + 1 < _NCHUNK)
                    def _():
                        idx_drain(c + 1, 1 - b, (b3 + 1) % 3)

                    @pl.when(c >= 1)
                    def _():
                        sc_drain(1 - b, (b3 + 2) % 3)

                    @pl.when(c + 1 < _NCHUNK)
                    def _():
                        gat_issue(l, 1 - b)
                    gat_drain(l, b)

                    @plsc.parallel_loop(0, _CK, unroll=8)
                    def _(jr):
                        eb = plsc.load_gather(
                            ebuf[b], [jnp.full((16,), jr, jnp.int32)])
                        for q in range(8):
                            rows[b][jr, pl.ds(q * 16, 16)] = (
                                rows[b][jr, pl.ds(q * 16, 16)] * eb)
                    sc_issue(b, b3)

                    @pl.when(c + 2 < _NCHUNK)
                    def _():
                        idx_issue(c + 2, b, (b3 + 2) % 3)
                return bcarry
            lax.fori_loop(0, _NCHUNK // 6, six, 0)
            sc_drain(1, 2)
            plsc.subcore_barrier()

            @pl.when(cid == 0)
            def _():
                pltpu.sync_copy(sp_out.at[pl.ds(rbase, _NROW)],
                                og_h.at[l, 0, pl.ds(rbase, _NROW)])
                pltpu.sync_copy(sp_den.at[pl.ds(rbase, _NROW)], v_den)
                pltpu.sync_copy(v_den, den_h.at[pl.ds(l * _N + rbase, _NROW)])

            @pl.when(cid == 1)
            def _():
                pltpu.sync_copy(sp_out.at[pl.ds(rbase, _NROW)],
                                og_h.at[l, 1, pl.ds(rbase, _NROW)])

            @pl.when((sid == 0) & (cid == 0))
            def _():
                pltpu.sync_copy(sp_out.at[pl.ds(16 * _NROW, _NTAIL)],
                                og_h.at[l, 0, pl.ds(16 * _NROW, _NTAIL)])
                pltpu.sync_copy(sp_den.at[pl.ds(16 * _NROW, _NTAIL)],
                                v_den.at[pl.ds(0, _NTAIL)])
                pltpu.sync_copy(v_den.at[pl.ds(0, _NTAIL)],
                                den_h.at[pl.ds(l * _N + 16 * _NROW, _NTAIL)])

            @pl.when((sid == 0) & (cid == 1))
            def _():
                pltpu.sync_copy(sp_out.at[pl.ds(16 * _NROW, _NTAIL)],
                                og_h.at[l, 1, pl.ds(16 * _NROW, _NTAIL)])

            plsc.subcore_barrier()
            return carry
        lax.fori_loop(0, _LAYERS, layer, 0)

    return k(src, dst, aa_s, aa_d, h, z_rows)


def kernel(x, edge_index, lin1_W, lin1_b, gat_W, att_src, att_dst, gat_b,
           lstm_Wih, lstm_Whh, lin2_W, lin2_b):
    loop = jnp.arange(_N, dtype=edge_index.dtype)
    pad = jnp.zeros((_EPAD - _E1,), dtype=edge_index.dtype)
    src = jnp.concatenate([edge_index[0], loop, pad])
    dst = jnp.concatenate([edge_index[1], loop, pad])
    x0, h, aa = _run_a(x, lin1_W, lin1_b, gat_W, att_src, att_dst)
    aa_s = jnp.transpose(aa[:, 0::2])
    aa_d = jnp.transpose(aa[:, 1::2])
    z_rows = jnp.zeros((_CK, 128), jnp.float32)
    og, den = _sc_edges(src, dst, aa_s.reshape(-1), aa_d.reshape(-1), h,
                        z_rows)
    den = den.reshape(_LAYERS, _N)
    wihT = jnp.transpose(lstm_Wih, (0, 2, 1))
    whhT = jnp.transpose(lstm_Whh, (0, 2, 1))
    return _run_c(x0, og, jnp.transpose(den), gat_b, wihT, whhT, lin2_W, lin2_b)


# speedup vs baseline: 1.0703x; 1.0049x over previous
import functools

import jax
import jax.numpy as jnp
from jax import lax
from jax.experimental import pallas as pl
from jax.experimental.pallas import tpu as pltpu
from jax.experimental.pallas import tpu_sc as plsc

_N = 10000
_E = 320000
_IN_DIM = 128
_OUT_DIM = 128
_DIM = 256
_HID = 256
_LAYERS = 4

_BN = 1000


def _kernel_a(x_ref, w1_ref, b1_ref, wg_ref, asrc_ref, adst_ref,
              x0_ref, h_ref, aa_ref):
    x0 = jnp.dot(x_ref[...], w1_ref[...],
                 preferred_element_type=jnp.float32) + b1_ref[...]
    x0_ref[...] = x0
    cols = []
    for l in range(_LAYERS):
        h = jnp.dot(x0, wg_ref[l], preferred_element_type=jnp.float32)
        h_ref[l, 0] = h[:, :128]
        h_ref[l, 1] = h[:, 128:]
        cols.append(jnp.sum(h * asrc_ref[l][None, :], axis=1))
        cols.append(jnp.sum(h * adst_ref[l][None, :], axis=1))
    aa_ref[...] = jnp.stack(cols, axis=1)


def _run_a(x, lin1_W, lin1_b, gat_W, att_src, att_dst):
    nblk = _N // _BN
    full = lambda *shape: pl.BlockSpec(shape, lambda i: (0,) * len(shape))
    return pl.pallas_call(
        _kernel_a,
        grid=(nblk,),
        in_specs=[
            pl.BlockSpec((_BN, _IN_DIM), lambda i: (i, 0)),
            full(_IN_DIM, _DIM),
            full(1, _DIM),
            full(_LAYERS, _DIM, _DIM),
            full(_LAYERS, _DIM),
            full(_LAYERS, _DIM),
        ],
        out_specs=[
            pl.BlockSpec((_BN, _DIM), lambda i: (i, 0)),
            pl.BlockSpec((_LAYERS, 2, _BN, 128), lambda i: (0, 0, i, 0)),
            pl.BlockSpec((_BN, 2 * _LAYERS), lambda i: (i, 0)),
        ],
        out_shape=[
            jax.ShapeDtypeStruct((_N, _DIM), jnp.float32),
            jax.ShapeDtypeStruct((_LAYERS, 2, _N, 128), jnp.float32),
            jax.ShapeDtypeStruct((_N, 2 * _LAYERS), jnp.float32),
        ],
    )(x, lin1_W, lin1_b.reshape(1, _DIM), gat_W, att_src, att_dst)


def _kernel_c(x0_ref, og_ref, den_ref, gb_ref, wih_ref, whh_ref,
              w2_ref, b2_ref, out_ref):
    xc = x0_ref[...]
    h = jnp.zeros((_BN, _HID), dtype=jnp.float32)
    c = jnp.zeros((_BN, _HID), dtype=jnp.float32)
    for l in range(_LAYERS):
        og = jnp.concatenate([og_ref[l, 0], og_ref[l, 1]], axis=1)
        den = den_ref[:, l:l + 1] + 1e-16
        ht = jnp.tanh(og / den + gb_ref[l][None, :])
        incat = jnp.concatenate([ht, xc], axis=1)
        dn = (((1,), (1,)), ((), ()))
        gates = (lax.dot_general(incat, wih_ref[l], dn,
                                 preferred_element_type=jnp.float32)
                 + lax.dot_general(h, whh_ref[l], dn,
                                   preferred_element_type=jnp.float32))
        gi = jax.nn.sigmoid(gates[:, 0:_HID])
        gf = jax.nn.sigmoid(gates[:, _HID:2 * _HID])
        gg = jnp.tanh(gates[:, 2 * _HID:3 * _HID])
        go = jax.nn.sigmoid(gates[:, 3 * _HID:4 * _HID])
        c = gf * c + gi * gg
        h = go * jnp.tanh(c)
        xc = h
    out_ref[...] = jnp.dot(xc, w2_ref[...],
                           preferred_element_type=jnp.float32) + b2_ref[...]


def _run_c(x0, og, den, gat_b, wihT, whhT, lin2_W, lin2_b):
    nblk = _N // _BN
    full = lambda *shape: pl.BlockSpec(shape, lambda i: (0,) * len(shape))
    return pl.pallas_call(
        _kernel_c,
        grid=(nblk,),
        in_specs=[
            pl.BlockSpec((_BN, _DIM), lambda i: (i, 0)),
            pl.BlockSpec((_LAYERS, 2, _BN, 128), lambda i: (0, 0, i, 0)),
            pl.BlockSpec((_BN, _LAYERS), lambda i: (i, 0)),
            full(_LAYERS, _DIM),
            full(_LAYERS, 4 * _HID, 2 * _DIM),
            full(_LAYERS, 4 * _HID, _HID),
            full(_DIM, _OUT_DIM),
            full(1, _OUT_DIM),
        ],
        out_specs=pl.BlockSpec((_BN, _OUT_DIM), lambda i: (i, 0)),
        out_shape=jax.ShapeDtypeStruct((_N, _OUT_DIM), jnp.float32),
    )(x0, og, den, gat_b, wihT, whhT, lin2_W, lin2_b.reshape(1, _OUT_DIM))


_CK = 96
_NCHUNK = 216
_EP_TILE = _NCHUNK * _CK
_EPAD = 16 * _EP_TILE
_E1 = _E + _N
_NROW = 624
_NTAIL = _N - 16 * _NROW


def _sc_edges(src, dst, aa_s, aa_d, h, z_rows):
    mesh = plsc.VectorSubcoreMesh(core_axis_name="c", subcore_axis_name="s")

    @functools.partial(
        pl.kernel,
        mesh=mesh,
        out_type=[
            jax.ShapeDtypeStruct((_LAYERS, 2, _N, 128), jnp.float32),
            jax.ShapeDtypeStruct((_LAYERS * _N,), jnp.float32),
        ],
        scratch_types=[
            pltpu.VMEM_SHARED((_N, 128), jnp.float32),
            pltpu.VMEM_SHARED((_N,), jnp.float32),
            pltpu.VMEM((_N,), jnp.float32),
            pltpu.VMEM((_N,), jnp.float32),
            pltpu.VMEM((_CK,), jnp.int32),
            pltpu.VMEM((_CK,), jnp.int32),
            pltpu.VMEM((_CK,), jnp.int32),
            pltpu.VMEM((_CK,), jnp.int32),
            pltpu.VMEM((_CK,), jnp.int32),
            pltpu.VMEM((_CK, 128), jnp.float32),
            pltpu.VMEM((_CK, 128), jnp.float32),
            pltpu.VMEM((_CK,), jnp.float32),
            pltpu.VMEM((_CK,), jnp.float32),
            pltpu.VMEM((_NROW,), jnp.float32),
            pltpu.SemaphoreType.DMA,
            pltpu.SemaphoreType.DMA,
            pltpu.SemaphoreType.DMA,
        ],
        compiler_params=pltpu.CompilerParams(needs_layout_passes=False),
    )
    def k(src_h, dst_h, aas_h, aad_h, h_h, zr_h, og_h, den_h,
          sp_out, sp_den, v_as, v_ad, v_src0, v_src1, v_dst0, v_dst1,
          v_dst2, v_rows0, v_rows1, v_e0, v_e1, v_den, sem_g, sem_i, sem_s):
        cid = lax.axis_index("c")
        sid = lax.axis_index("s")
        rbase = sid * _NROW
        ebase = sid * _EP_TILE
        rows = (v_rows0, v_rows1)
        sbuf = (v_src0, v_src1)
        dbuf = (v_dst0, v_dst1, v_dst2)
        ebuf = (v_e0, v_e1)

        def idx_issue(c, b, b3):
            off = ebase + c * _CK
            pltpu.async_copy(src_h.at[pl.ds(off, _CK)], sbuf[b], sem_i)
            pltpu.async_copy(dst_h.at[pl.ds(off, _CK)], dbuf[b3], sem_i)

        def idx_drain(c, b, b3):
            off = ebase + c * _CK
            pltpu.make_async_copy(src_h.at[pl.ds(off, _CK)], sbuf[b],
                                  sem_i).wait()
            pltpu.make_async_copy(dst_h.at[pl.ds(off, _CK)], dbuf[b3],
                                  sem_i).wait()

        def sc_issue(b, b3):
            pltpu.async_copy(rows[b], sp_out.at[dbuf[b3]], sem_s, add=True)
            pltpu.async_copy(ebuf[b], sp_den.at[dbuf[b3]], sem_s, add=True)

        def sc_drain(b, b3):
            pltpu.make_async_copy(rows[b], sp_out.at[dbuf[b3]],
                                  sem_s).wait()
            pltpu.make_async_copy(ebuf[b], sp_den.at[dbuf[b3]],
                                  sem_s).wait()

        def gat_issue(l, b):
            @pl.when(cid == 0)
            def _():
                pltpu.async_copy(h_h.at[l].at[0].at[sbuf[b]], rows[b], sem_g)

            @pl.when(cid == 1)
            def _():
                pltpu.async_copy(h_h.at[l].at[1].at[sbuf[b]], rows[b], sem_g)

        def gat_drain(l, b):
            pltpu.make_async_copy(h_h.at[l].at[0].at[sbuf[b]], rows[b],
                                  sem_g).wait()

        def layer(l, carry):
            pltpu.sync_copy(aas_h.at[pl.ds(l * _N, _N)], v_as)
            pltpu.sync_copy(aad_h.at[pl.ds(l * _N, _N)], v_ad)
            pltpu.sync_copy(zr_h, v_rows0)
            for p in range(6):
                pltpu.sync_copy(v_rows0,
                                sp_out.at[pl.ds(rbase + p * 96, 96)])
            pltpu.sync_copy(v_rows0.at[pl.ds(0, 48)],
                            sp_out.at[pl.ds(rbase + 576, 48)])
            for p in range(4):
                pltpu.sync_copy(v_rows0.at[p],
                                sp_den.at[pl.ds(rbase + p * 128, 128)])
            pltpu.sync_copy(v_rows0.at[4, pl.ds(0, 112)],
                            sp_den.at[pl.ds(rbase + 512, 112)])

            @pl.when(sid == 0)
            def _():
                pltpu.sync_copy(v_rows0.at[pl.ds(0, _NTAIL)],
                                sp_out.at[pl.ds(16 * _NROW, _NTAIL)])
                pltpu.sync_copy(v_rows0.at[5, pl.ds(0, _NTAIL)],
                                sp_den.at[pl.ds(16 * _NROW, _NTAIL)])

            plsc.subcore_barrier()

            idx_issue(0, 0, 0)
            idx_drain(0, 0, 0)
            idx_issue(1, 1, 1)
            gat_issue(l, 0)

            def six(kk, bcarry):
                for b6 in range(6):
                    c = kk * 6 + b6
                    b = b6 % 2
                    b3 = b6 % 3
                    off = ebase + c * _CK
                    for g in range(_CK // 16):
                        sidx = sbuf[b][pl.ds(g * 16, 16)]
                        didx = dbuf[b3][pl.ds(g * 16, 16)]
                        a = (plsc.load_gather(v_as, [sidx])
                             + plsc.load_gather(v_ad, [didx]))
                        a = jnp.where(a > 0, a, 0.2 * a)
                        e = jnp.exp(a)
                        gi = off + g * 16 + lax.iota(jnp.int32, 16)
                        ebuf[b][pl.ds(g * 16, 16)] = jnp.where(
                            gi < _E1, e, 0.0)

                    @pl.when(c + 1 < _NCHUNK)
                    def _():
                        idx_drain(c + 1, 1 - b, (b3 + 1) % 3)

                    @pl.when(c >= 1)
                    def _():
                        sc_drain(1 - b, (b3 + 2) % 3)

                    @pl.when(c + 1 < _NCHUNK)
                    def _():
                        gat_issue(l, 1 - b)
                    gat_drain(l, b)

                    @plsc.parallel_loop(0, _CK, unroll=8)
                    def _(jr):
                        eb = plsc.load_gather(
                            ebuf[b], [jnp.full((16,), jr, jnp.int32)])
                        for q in range(8):
                            rows[b][jr, pl.ds(q * 16, 16)] = (
                                rows[b][jr, pl.ds(q * 16, 16)] * eb)
                    sc_issue(b, b3)

                    @pl.when(c + 2 < _NCHUNK)
                    def _():
                        idx_issue(c + 2, b, (b3 + 2) % 3)
                return bcarry
            lax.fori_loop(0, _NCHUNK // 6, six, 0)
            sc_drain(1, 2)
            plsc.subcore_barrier()

            @pl.when(cid == 0)
            def _():
                pltpu.sync_copy(sp_out.at[pl.ds(rbase, _NROW)],
                                og_h.at[l, 0, pl.ds(rbase, _NROW)])
                pltpu.sync_copy(sp_den.at[pl.ds(rbase, _NROW)], v_den)
                pltpu.sync_copy(v_den, den_h.at[pl.ds(l * _N + rbase, _NROW)])

            @pl.when(cid == 1)
            def _():
                pltpu.sync_copy(sp_out.at[pl.ds(rbase, _NROW)],
                                og_h.at[l, 1, pl.ds(rbase, _NROW)])

            @pl.when((sid == 0) & (cid == 0))
            def _():
                pltpu.sync_copy(sp_out.at[pl.ds(16 * _NROW, _NTAIL)],
                                og_h.at[l, 0, pl.ds(16 * _NROW, _NTAIL)])
                pltpu.sync_copy(sp_den.at[pl.ds(16 * _NROW, _NTAIL)],
                                v_den.at[pl.ds(0, _NTAIL)])
                pltpu.sync_copy(v_den.at[pl.ds(0, _NTAIL)],
                                den_h.at[pl.ds(l * _N + 16 * _NROW, _NTAIL)])

            @pl.when((sid == 0) & (cid == 1))
            def _():
                pltpu.sync_copy(sp_out.at[pl.ds(16 * _NROW, _NTAIL)],
                                og_h.at[l, 1, pl.ds(16 * _NROW, _NTAIL)])

            plsc.subcore_barrier()
            return carry
        lax.fori_loop(0, _LAYERS, layer, 0)

    return k(src, dst, aa_s, aa_d, h, z_rows)


def kernel(x, edge_index, lin1_W, lin1_b, gat_W, att_src, att_dst, gat_b,
           lstm_Wih, lstm_Whh, lin2_W, lin2_b):
    loop = jnp.arange(_N, dtype=edge_index.dtype)
    pad = jnp.zeros((_EPAD - _E1,), dtype=edge_index.dtype)
    src = jnp.concatenate([edge_index[0], loop, pad])
    dst = jnp.concatenate([edge_index[1], loop, pad])
    x0, h, aa = _run_a(x, lin1_W, lin1_b, gat_W, att_src, att_dst)
    aa_s = jnp.transpose(aa[:, 0::2])
    aa_d = jnp.transpose(aa[:, 1::2])
    z_rows = jnp.zeros((_CK, 128), jnp.float32)
    og, den = _sc_edges(src, dst, aa_s.reshape(-1), aa_d.reshape(-1), h,
                        z_rows)
    den = den.reshape(_LAYERS, _N)
    return _run_c(x0, og, jnp.transpose(den), gat_b, lstm_Wih, lstm_Whh,
                  lin2_W, lin2_b)


# speedup vs baseline: 1.0722x; 1.0018x over previous
import functools

import jax
import jax.numpy as jnp
from jax import lax
from jax.experimental import pallas as pl
from jax.experimental.pallas import tpu as pltpu
from jax.experimental.pallas import tpu_sc as plsc

_N = 10000
_E = 320000
_IN_DIM = 128
_OUT_DIM = 128
_DIM = 256
_HID = 256
_LAYERS = 4

_BN = 2000


def _kernel_a(x_ref, w1_ref, b1_ref, wg_ref, asrc_ref, adst_ref,
              x0_ref, h_ref, aa_ref):
    x0 = jnp.dot(x_ref[...], w1_ref[...],
                 preferred_element_type=jnp.float32) + b1_ref[...]
    x0_ref[...] = x0
    cols = []
    for l in range(_LAYERS):
        h = jnp.dot(x0, wg_ref[l], preferred_element_type=jnp.float32)
        h_ref[l, 0] = h[:, :128]
        h_ref[l, 1] = h[:, 128:]
        cols.append(jnp.sum(h * asrc_ref[l][None, :], axis=1))
        cols.append(jnp.sum(h * adst_ref[l][None, :], axis=1))
    aa_ref[...] = jnp.stack(cols, axis=1)


def _run_a(x, lin1_W, lin1_b, gat_W, att_src, att_dst):
    nblk = _N // _BN
    full = lambda *shape: pl.BlockSpec(shape, lambda i: (0,) * len(shape))
    return pl.pallas_call(
        _kernel_a,
        grid=(nblk,),
        in_specs=[
            pl.BlockSpec((_BN, _IN_DIM), lambda i: (i, 0)),
            full(_IN_DIM, _DIM),
            full(1, _DIM),
            full(_LAYERS, _DIM, _DIM),
            full(_LAYERS, _DIM),
            full(_LAYERS, _DIM),
        ],
        out_specs=[
            pl.BlockSpec((_BN, _DIM), lambda i: (i, 0)),
            pl.BlockSpec((_LAYERS, 2, _BN, 128), lambda i: (0, 0, i, 0)),
            pl.BlockSpec((_BN, 2 * _LAYERS), lambda i: (i, 0)),
        ],
        out_shape=[
            jax.ShapeDtypeStruct((_N, _DIM), jnp.float32),
            jax.ShapeDtypeStruct((_LAYERS, 2, _N, 128), jnp.float32),
            jax.ShapeDtypeStruct((_N, 2 * _LAYERS), jnp.float32),
        ],
    )(x, lin1_W, lin1_b.reshape(1, _DIM), gat_W, att_src, att_dst)


def _kernel_c(x0_ref, og_ref, den_ref, gb_ref, wih_ref, whh_ref,
              w2_ref, b2_ref, out_ref):
    xc = x0_ref[...]
    h = jnp.zeros((_BN, _HID), dtype=jnp.float32)
    c = jnp.zeros((_BN, _HID), dtype=jnp.float32)
    for l in range(_LAYERS):
        og = jnp.concatenate([og_ref[l, 0], og_ref[l, 1]], axis=1)
        den = den_ref[:, l:l + 1] + 1e-16
        ht = jnp.tanh(og / den + gb_ref[l][None, :])
        incat = jnp.concatenate([ht, xc], axis=1)
        dn = (((1,), (1,)), ((), ()))
        gates = (lax.dot_general(incat, wih_ref[l], dn,
                                 preferred_element_type=jnp.float32)
                 + lax.dot_general(h, whh_ref[l], dn,
                                   preferred_element_type=jnp.float32))
        gi = jax.nn.sigmoid(gates[:, 0:_HID])
        gf = jax.nn.sigmoid(gates[:, _HID:2 * _HID])
        gg = jnp.tanh(gates[:, 2 * _HID:3 * _HID])
        go = jax.nn.sigmoid(gates[:, 3 * _HID:4 * _HID])
        c = gf * c + gi * gg
        h = go * jnp.tanh(c)
        xc = h
    out_ref[...] = jnp.dot(xc, w2_ref[...],
                           preferred_element_type=jnp.float32) + b2_ref[...]


def _run_c(x0, og, den, gat_b, wihT, whhT, lin2_W, lin2_b):
    nblk = _N // _BN
    full = lambda *shape: pl.BlockSpec(shape, lambda i: (0,) * len(shape))
    return pl.pallas_call(
        _kernel_c,
        grid=(nblk,),
        in_specs=[
            pl.BlockSpec((_BN, _DIM), lambda i: (i, 0)),
            pl.BlockSpec((_LAYERS, 2, _BN, 128), lambda i: (0, 0, i, 0)),
            pl.BlockSpec((_BN, _LAYERS), lambda i: (i, 0)),
            full(_LAYERS, _DIM),
            full(_LAYERS, 4 * _HID, 2 * _DIM),
            full(_LAYERS, 4 * _HID, _HID),
            full(_DIM, _OUT_DIM),
            full(1, _OUT_DIM),
        ],
        out_specs=pl.BlockSpec((_BN, _OUT_DIM), lambda i: (i, 0)),
        out_shape=jax.ShapeDtypeStruct((_N, _OUT_DIM), jnp.float32),
    )(x0, og, den, gat_b, wihT, whhT, lin2_W, lin2_b.reshape(1, _OUT_DIM))


_CK = 96
_NCHUNK = 216
_EP_TILE = _NCHUNK * _CK
_EPAD = 16 * _EP_TILE
_E1 = _E + _N
_NROW = 624
_NTAIL = _N - 16 * _NROW


def _sc_edges(src, dst, aa_s, aa_d, h, z_rows):
    mesh = plsc.VectorSubcoreMesh(core_axis_name="c", subcore_axis_name="s")

    @functools.partial(
        pl.kernel,
        mesh=mesh,
        out_type=[
            jax.ShapeDtypeStruct((_LAYERS, 2, _N, 128), jnp.float32),
            jax.ShapeDtypeStruct((_LAYERS * _N,), jnp.float32),
        ],
        scratch_types=[
            pltpu.VMEM_SHARED((_N, 128), jnp.float32),
            pltpu.VMEM_SHARED((_N,), jnp.float32),
            pltpu.VMEM((_N,), jnp.float32),
            pltpu.VMEM((_N,), jnp.float32),
            pltpu.VMEM((_CK,), jnp.int32),
            pltpu.VMEM((_CK,), jnp.int32),
            pltpu.VMEM((_CK,), jnp.int32),
            pltpu.VMEM((_CK,), jnp.int32),
            pltpu.VMEM((_CK,), jnp.int32),
            pltpu.VMEM((_CK, 128), jnp.float32),
            pltpu.VMEM((_CK, 128), jnp.float32),
            pltpu.VMEM((_CK,), jnp.float32),
            pltpu.VMEM((_CK,), jnp.float32),
            pltpu.VMEM((_NROW,), jnp.float32),
            pltpu.SemaphoreType.DMA,
            pltpu.SemaphoreType.DMA,
            pltpu.SemaphoreType.DMA,
        ],
        compiler_params=pltpu.CompilerParams(needs_layout_passes=False),
    )
    def k(src_h, dst_h, aas_h, aad_h, h_h, zr_h, og_h, den_h,
          sp_out, sp_den, v_as, v_ad, v_src0, v_src1, v_dst0, v_dst1,
          v_dst2, v_rows0, v_rows1, v_e0, v_e1, v_den, sem_g, sem_i, sem_s):
        cid = lax.axis_index("c")
        sid = lax.axis_index("s")
        rbase = sid * _NROW
        ebase = sid * _EP_TILE
        rows = (v_rows0, v_rows1)
        sbuf = (v_src0, v_src1)
        dbuf = (v_dst0, v_dst1, v_dst2)
        ebuf = (v_e0, v_e1)

        def idx_issue(c, b, b3):
            off = ebase + c * _CK
            pltpu.async_copy(src_h.at[pl.ds(off, _CK)], sbuf[b], sem_i)
            pltpu.async_copy(dst_h.at[pl.ds(off, _CK)], dbuf[b3], sem_i)

        def idx_drain(c, b, b3):
            off = ebase + c * _CK
            pltpu.make_async_copy(src_h.at[pl.ds(off, _CK)], sbuf[b],
                                  sem_i).wait()
            pltpu.make_async_copy(dst_h.at[pl.ds(off, _CK)], dbuf[b3],
                                  sem_i).wait()

        def sc_issue(b, b3):
            pltpu.async_copy(rows[b], sp_out.at[dbuf[b3]], sem_s, add=True)
            pltpu.async_copy(ebuf[b], sp_den.at[dbuf[b3]], sem_s, add=True)

        def sc_drain(b, b3):
            pltpu.make_async_copy(rows[b], sp_out.at[dbuf[b3]],
                                  sem_s).wait()
            pltpu.make_async_copy(ebuf[b], sp_den.at[dbuf[b3]],
                                  sem_s).wait()

        def gat_issue(l, b):
            @pl.when(cid == 0)
            def _():
                pltpu.async_copy(h_h.at[l].at[0].at[sbuf[b]], rows[b], sem_g)

            @pl.when(cid == 1)
            def _():
                pltpu.async_copy(h_h.at[l].at[1].at[sbuf[b]], rows[b], sem_g)

        def gat_drain(l, b):
            pltpu.make_async_copy(h_h.at[l].at[0].at[sbuf[b]], rows[b],
                                  sem_g).wait()

        def layer(l, carry):
            pltpu.sync_copy(aas_h.at[pl.ds(l * _N, _N)], v_as)
            pltpu.sync_copy(aad_h.at[pl.ds(l * _N, _N)], v_ad)
            pltpu.sync_copy(zr_h, v_rows0)
            for p in range(6):
                pltpu.sync_copy(v_rows0,
                                sp_out.at[pl.ds(rbase + p * 96, 96)])
            pltpu.sync_copy(v_rows0.at[pl.ds(0, 48)],
                            sp_out.at[pl.ds(rbase + 576, 48)])
            for p in range(4):
                pltpu.sync_copy(v_rows0.at[p],
                                sp_den.at[pl.ds(rbase + p * 128, 128)])
            pltpu.sync_copy(v_rows0.at[4, pl.ds(0, 112)],
                            sp_den.at[pl.ds(rbase + 512, 112)])

            @pl.when(sid == 0)
            def _():
                pltpu.sync_copy(v_rows0.at[pl.ds(0, _NTAIL)],
                                sp_out.at[pl.ds(16 * _NROW, _NTAIL)])
                pltpu.sync_copy(v_rows0.at[5, pl.ds(0, _NTAIL)],
                                sp_den.at[pl.ds(16 * _NROW, _NTAIL)])

            plsc.subcore_barrier()

            idx_issue(0, 0, 0)
            idx_drain(0, 0, 0)
            idx_issue(1, 1, 1)
            gat_issue(l, 0)

            def six(kk, bcarry):
                for b6 in range(6):
                    c = kk * 6 + b6
                    b = b6 % 2
                    b3 = b6 % 3
                    off = ebase + c * _CK
                    for g in range(_CK // 16):
                        sidx = sbuf[b][pl.ds(g * 16, 16)]
                        didx = dbuf[b3][pl.ds(g * 16, 16)]
                        a = (plsc.load_gather(v_as, [sidx])
                             + plsc.load_gather(v_ad, [didx]))
                        a = jnp.where(a > 0, a, 0.2 * a)
                        e = jnp.exp(a)
                        gi = off + g * 16 + lax.iota(jnp.int32, 16)
                        ebuf[b][pl.ds(g * 16, 16)] = jnp.where(
                            gi < _E1, e, 0.0)

                    @pl.when(c + 1 < _NCHUNK)
                    def _():
                        idx_drain(c + 1, 1 - b, (b3 + 1) % 3)

                    @pl.when(c >= 1)
                    def _():
                        sc_drain(1 - b, (b3 + 2) % 3)

                    @pl.when(c + 1 < _NCHUNK)
                    def _():
                        gat_issue(l, 1 - b)
                    gat_drain(l, b)

                    @plsc.parallel_loop(0, _CK, unroll=8)
                    def _(jr):
                        eb = plsc.load_gather(
                            ebuf[b], [jnp.full((16,), jr, jnp.int32)])
                        for q in range(8):
                            rows[b][jr, pl.ds(q * 16, 16)] = (
                                rows[b][jr, pl.ds(q * 16, 16)] * eb)
                    sc_issue(b, b3)

                    @pl.when(c + 2 < _NCHUNK)
                    def _():
                        idx_issue(c + 2, b, (b3 + 2) % 3)
                return bcarry
            lax.fori_loop(0, _NCHUNK // 6, six, 0)
            sc_drain(1, 2)
            plsc.subcore_barrier()

            @pl.when(cid == 0)
            def _():
                pltpu.sync_copy(sp_out.at[pl.ds(rbase, _NROW)],
                                og_h.at[l, 0, pl.ds(rbase, _NROW)])
                pltpu.sync_copy(sp_den.at[pl.ds(rbase, _NROW)], v_den)
                pltpu.sync_copy(v_den, den_h.at[pl.ds(l * _N + rbase, _NROW)])

            @pl.when(cid == 1)
            def _():
                pltpu.sync_copy(sp_out.at[pl.ds(rbase, _NROW)],
                                og_h.at[l, 1, pl.ds(rbase, _NROW)])

            @pl.when((sid == 0) & (cid == 0))
            def _():
                pltpu.sync_copy(sp_out.at[pl.ds(16 * _NROW, _NTAIL)],
                                og_h.at[l, 0, pl.ds(16 * _NROW, _NTAIL)])
                pltpu.sync_copy(sp_den.at[pl.ds(16 * _NROW, _NTAIL)],
                                v_den.at[pl.ds(0, _NTAIL)])
                pltpu.sync_copy(v_den.at[pl.ds(0, _NTAIL)],
                                den_h.at[pl.ds(l * _N + 16 * _NROW, _NTAIL)])

            @pl.when((sid == 0) & (cid == 1))
            def _():
                pltpu.sync_copy(sp_out.at[pl.ds(16 * _NROW, _NTAIL)],
                                og_h.at[l, 1, pl.ds(16 * _NROW, _NTAIL)])

            plsc.subcore_barrier()
            return carry
        lax.fori_loop(0, _LAYERS, layer, 0)

    return k(src, dst, aa_s, aa_d, h, z_rows)


def kernel(x, edge_index, lin1_W, lin1_b, gat_W, att_src, att_dst, gat_b,
           lstm_Wih, lstm_Whh, lin2_W, lin2_b):
    loop = jnp.arange(_N, dtype=edge_index.dtype)
    pad = jnp.zeros((_EPAD - _E1,), dtype=edge_index.dtype)
    src = jnp.concatenate([edge_index[0], loop, pad])
    dst = jnp.concatenate([edge_index[1], loop, pad])
    x0, h, aa = _run_a(x, lin1_W, lin1_b, gat_W, att_src, att_dst)
    aa_s = jnp.transpose(aa[:, 0::2])
    aa_d = jnp.transpose(aa[:, 1::2])
    z_rows = jnp.zeros((_CK, 128), jnp.float32)
    og, den = _sc_edges(src, dst, aa_s.reshape(-1), aa_d.reshape(-1), h,
                        z_rows)
    den = den.reshape(_LAYERS, _N)
    return _run_c(x0, og, jnp.transpose(den), gat_b, lstm_Wih, lstm_Whh,
                  lin2_W, lin2_b)
